# baseline (speedup 1.0000x reference)
import functools

import jax
import jax.numpy as jnp
from jax import lax
from jax.experimental import pallas as pl
from jax.experimental.pallas import tpu as pltpu
from jax.experimental.pallas import tpu_sc as plsc

_NC, _NS, _L = 2, 16, 16
_NW = _NC * _NS
_R, _C = 4096, 512
_V = _C // _L
_RPW = _R // _NW
_RB = 32
_NBLK = _RPW // _RB


def _row_score(ks, wbuf, num_vec, den_vec):
    for i in range(_V):
        d = (i % 2 == 0)
        ks[i] = plsc.sort_key_val(ks[i], ks[i], descending=d)[0]
    for K in (2, 4, 8, 16, 32):
        J = K // 2
        while J >= 1:
            for b in range(_V):
                if b & J == 0:
                    q = b | J
                    ka, kb = ks[b], ks[q]
                    lo = jnp.minimum(ka, kb)
                    hi = jnp.maximum(ka, kb)
                    if (b & K) == 0:
                        ks[b], ks[q] = hi, lo
                    else:
                        ks[b], ks[q] = lo, hi
            J //= 2
        for i in range(_V):
            d = (i & K) == 0
            ks[i] = plsc.sort_key_val(ks[i], ks[i], descending=d)[0]
    carry = jnp.zeros((_L,), jnp.float32)
    one = jnp.int32(1)
    for i in range(_V):
        lbit = lax.bitcast_convert_type(ks[i], jnp.int32) & one
        sl = lbit.astype(jnp.float32)
        pre = plsc.cumsum(sl)
        w = wbuf[pl.ds(i * _L, _L)]
        num_vec = num_vec + sl * (carry + pre) * w
        cnt = plsc.all_reduce_population_count(lbit != 0)
        carry = carry + cnt.astype(jnp.float32)
    return num_vec, den_vec + carry


def _sc_body(preds_hbm, labels_hbm, out_hbm,
             pbufs, lbufs, wbuf, obuf, sems):
    wid = lax.axis_index("s") * _NC + lax.axis_index("c")
    iota_f = lax.iota(jnp.int32, _L).astype(jnp.float32)
    for i in range(_V):
        wbuf[pl.ds(i * _L, _L)] = 1.0 / (iota_f + float(i * _L + 1))
    row0 = wid * _RPW

    def copies(blk, slot):
        off = row0 + blk * _RB
        return (
            pltpu.make_async_copy(
                preds_hbm.at[pl.ds(off, _RB)], pbufs[slot], sems[slot]),
            pltpu.make_async_copy(
                labels_hbm.at[pl.ds(off, _RB)], lbufs[slot], sems[slot]),
        )

    def row_body_for(pbuf, lbuf):
        def row_body(r, carry2):
            ks = []
            for i in range(_V):
                p = pbuf[r, pl.ds(i * _L, _L)]
                l = lbuf[r, pl.ds(i * _L, _L)]
                s = lax.bitcast_convert_type(p, jnp.int32)
                li = l.astype(jnp.int32)
                k = (s & jnp.int32(-2)) | li
                ks.append(lax.bitcast_convert_type(k, jnp.float32))
            return _row_score(ks, wbuf, *carry2)
        return row_body

    for c in copies(0, 0):
        c.start()
    zeros = jnp.zeros((_L,), jnp.float32)
    carry = (zeros, zeros)
    for blk in range(_NBLK):
        slot = blk % 2
        if blk + 1 < _NBLK:
            nxt = copies(blk + 1, 1 - slot)
            for c in nxt:
                c.start()
        for c in copies(blk, slot):
            c.wait()
        carry = plsc.parallel_loop(0, _RB, 1, unroll=2, carry=carry)(
            row_body_for(pbufs[slot], lbufs[slot]))
    num_vec, den_vec = carry
    obuf[pl.ds(0, _L)] = num_vec
    obuf[pl.ds(_L, _L)] = den_vec * (1.0 / _L)
    pltpu.sync_copy(obuf, out_hbm.at[pl.ds(wid * 2 * _L, 2 * _L)])


def kernel(preds, labels):
    mesh = plsc.VectorSubcoreMesh(
        core_axis_name="c", subcore_axis_name="s",
        num_cores=_NC, num_subcores=_NS)
    k = functools.partial(
        pl.kernel,
        out_type=jax.ShapeDtypeStruct((_NW * 2 * _L,), jnp.float32),
        mesh=mesh,
        compiler_params=pltpu.CompilerParams(needs_layout_passes=False),
        scratch_types=[
            [pltpu.VMEM((_RB, _C), jnp.float32)] * 2,
            [pltpu.VMEM((_RB, _C), jnp.float32)] * 2,
            pltpu.VMEM((_C,), jnp.float32),
            pltpu.VMEM((2 * _L,), jnp.float32),
            [pltpu.SemaphoreType.DMA] * 2,
        ],
    )(_sc_body)
    out = k(preds, labels)
    o = out.reshape(_NW, 2, _L)
    return o[:, 0].sum() / o[:, 1].sum()


# speedup vs baseline: 1.0496x; 1.0496x over previous
import functools

import jax
import jax.numpy as jnp
from jax import lax
from jax.experimental import pallas as pl
from jax.experimental.pallas import tpu as pltpu
from jax.experimental.pallas import tpu_sc as plsc

_NC, _NS, _L = 2, 16, 16
_NW = _NC * _NS
_R, _C = 4096, 512
_V = _C // _L
_RPW = _R // _NW
_RB = 32
_NBLK = _RPW // _RB


def _row_score(ks, wbuf, num_vec, den_vec):
    for i in range(_V):
        d = (i % 2 == 0)
        ks[i] = plsc.sort_key_val(ks[i], ks[i], descending=d)[0]
    for K in (2, 4, 8, 16, 32):
        J = K // 2
        while J >= 1:
            for b in range(_V):
                if b & J == 0:
                    q = b | J
                    ka, kb = ks[b], ks[q]
                    lo = jnp.minimum(ka, kb)
                    hi = jnp.maximum(ka, kb)
                    if (b & K) == 0:
                        ks[b], ks[q] = hi, lo
                    else:
                        ks[b], ks[q] = lo, hi
            J //= 2
        for i in range(_V):
            d = (i & K) == 0
            ks[i] = plsc.sort_key_val(ks[i], ks[i], descending=d)[0]
    carry = jnp.zeros((_L,), jnp.float32)
    one = jnp.int32(1)
    for i in range(_V):
        lbit = lax.bitcast_convert_type(ks[i], jnp.int32) & one
        sl = lbit.astype(jnp.float32)
        pre = plsc.cumsum(sl)
        w = wbuf[pl.ds(i * _L, _L)]
        num_vec = num_vec + sl * (carry + pre) * w
        cnt = plsc.all_reduce_population_count(lbit != 0)
        carry = carry + cnt.astype(jnp.float32)
    return num_vec, den_vec + carry


def _sc_body(preds_hbm, labels_hbm, out_hbm,
             pbufs, lbufs, wbuf, obuf, sems):
    wid = lax.axis_index("s") * _NC + lax.axis_index("c")
    iota_f = lax.iota(jnp.int32, _L).astype(jnp.float32)
    for i in range(_V):
        wbuf[pl.ds(i * _L, _L)] = 1.0 / (iota_f + float(i * _L + 1))
    row0 = wid * _RPW

    def copies(blk, slot):
        off = row0 + blk * _RB
        return (
            pltpu.make_async_copy(
                preds_hbm.at[pl.ds(off, _RB)], pbufs[slot], sems[slot]),
            pltpu.make_async_copy(
                labels_hbm.at[pl.ds(off, _RB)], lbufs[slot], sems[slot]),
        )

    def row_body_for(pbuf, lbuf):
        def row_body(r, carry2):
            ks = []
            for i in range(_V):
                p = pbuf[r, pl.ds(i * _L, _L)]
                l = lbuf[r, pl.ds(i * _L, _L)]
                s = lax.bitcast_convert_type(p, jnp.int32)
                li = l.astype(jnp.int32)
                k = (s & jnp.int32(-2)) | li
                ks.append(lax.bitcast_convert_type(k, jnp.float32))
            return _row_score(ks, wbuf, *carry2)
        return row_body

    for c in copies(0, 0):
        c.start()
    zeros = jnp.zeros((_L,), jnp.float32)
    carry = (zeros, zeros)
    for blk in range(_NBLK):
        slot = blk % 2
        if blk + 1 < _NBLK:
            nxt = copies(blk + 1, 1 - slot)
            for c in nxt:
                c.start()
        for c in copies(blk, slot):
            c.wait()
        carry = plsc.parallel_loop(0, _RB, 1, unroll=1, carry=carry)(
            row_body_for(pbufs[slot], lbufs[slot]))
    num_vec, den_vec = carry
    obuf[pl.ds(0, _L)] = num_vec
    obuf[pl.ds(_L, _L)] = den_vec * (1.0 / _L)
    pltpu.sync_copy(obuf, out_hbm.at[pl.ds(wid * 2 * _L, 2 * _L)])


def kernel(preds, labels):
    mesh = plsc.VectorSubcoreMesh(
        core_axis_name="c", subcore_axis_name="s",
        num_cores=_NC, num_subcores=_NS)
    k = functools.partial(
        pl.kernel,
        out_type=jax.ShapeDtypeStruct((_NW * 2 * _L,), jnp.float32),
        mesh=mesh,
        compiler_params=pltpu.CompilerParams(needs_layout_passes=False),
        scratch_types=[
            [pltpu.VMEM((_RB, _C), jnp.float32)] * 2,
            [pltpu.VMEM((_RB, _C), jnp.float32)] * 2,
            pltpu.VMEM((_C,), jnp.float32),
            pltpu.VMEM((2 * _L,), jnp.float32),
            [pltpu.SemaphoreType.DMA] * 2,
        ],
    )(_sc_body)
    out = k(preds, labels)
    o = out.reshape(_NW, 2, _L)
    return o[:, 0].sum() / o[:, 1].sum()
